# Initial kernel scaffold; baseline (speedup 1.0000x reference)
import jax
import jax.numpy as jnp
from jax.experimental import pallas as pl


def kernel(x, W1, b1, W2, b2, Wmu, bmu, Wlv, blv, Wd1, bd1, Wd2, bd2, Wd3, bd3):
    raise NotImplementedError("write your pallas kernel here")


# speedup vs baseline: 1.6778x; 1.6778x over previous
import jax
import jax.numpy as jnp
from jax.experimental import pallas as pl

G = 5
E = 5
J = 5
T = 9
D = 12
IN = T * J * D
H1 = 512
H2 = 256
ZD = 64
B = 1024
BT = 512
NB = B // BT


def _moe_kernel(xf_ref, W1_ref, b1_ref, W2_ref, b2_ref, Wmu_ref, bmu_ref,
                Wlv_ref, blv_ref, Wd1_ref, bd1_ref, Wd2_ref, bd2_ref,
                Wd3_ref, bd3_ref, mu_ref, lv_ref, xh_ref, idx_ref):
    xfb = xf_ref[0]

    def expert(e):
        h1 = jax.nn.relu(jnp.dot(xfb, W1_ref[0, e]) + b1_ref[0, e])
        h2 = jax.nn.relu(jnp.dot(h1, W2_ref[0, e]) + b2_ref[0, e])
        mu = jnp.dot(h2, Wmu_ref[0, e]) + bmu_ref[0, e]
        lv = jnp.dot(h2, Wlv_ref[0, e]) + blv_ref[0, e]
        d1 = jax.nn.relu(jnp.dot(mu, Wd1_ref[0, e]) + bd1_ref[0, e])
        d2 = jax.nn.relu(jnp.dot(d1, Wd2_ref[0, e]) + bd2_ref[0, e])
        xh = jnp.dot(d2, Wd3_ref[0, e]) + bd3_ref[0, e]
        diff = xh - xfb
        err = jnp.mean(diff * diff, axis=-1, keepdims=True)
        return mu, lv, xh, err

    mu_b, lv_b, xh_b, err_b = expert(0)
    idx_b = jnp.zeros((BT, 1), dtype=jnp.int32)
    for e in range(1, E):
        mu_e, lv_e, xh_e, err_e = expert(e)
        better = err_e < err_b
        mu_b = jnp.where(better, mu_e, mu_b)
        lv_b = jnp.where(better, lv_e, lv_b)
        xh_b = jnp.where(better, xh_e, xh_b)
        idx_b = jnp.where(better, jnp.int32(e), idx_b)
        err_b = jnp.where(better, err_e, err_b)

    mu_ref[0] = mu_b
    lv_ref[0] = lv_b
    xh_ref[0] = xh_b
    idx_ref[0] = idx_b


def kernel(x, W1, b1, W2, b2, Wmu, bmu, Wlv, blv, Wd1, bd1, Wd2, bd2, Wd3, bd3):
    Bb = x.shape[0]
    nb = Bb // BT
    xf = x.reshape(Bb, T, G, J, D).transpose(2, 0, 1, 3, 4).reshape(G, Bb, IN)

    wspec = lambda *s: pl.BlockSpec((1,) + s, lambda g, b: (g,) + (0,) * len(s))
    mu_sel, lv_sel, xh_sel, idx = pl.pallas_call(
        _moe_kernel,
        grid=(G, nb),
        in_specs=[
            pl.BlockSpec((1, BT, IN), lambda g, b: (g, b, 0)),
            wspec(E, IN, H1), wspec(E, H1),
            wspec(E, H1, H2), wspec(E, H2),
            wspec(E, H2, ZD), wspec(E, ZD),
            wspec(E, H2, ZD), wspec(E, ZD),
            wspec(E, ZD, H2), wspec(E, H2),
            wspec(E, H2, H1), wspec(E, H1),
            wspec(E, H1, IN), wspec(E, IN),
        ],
        out_specs=[
            pl.BlockSpec((1, BT, ZD), lambda g, b: (g, b, 0)),
            pl.BlockSpec((1, BT, ZD), lambda g, b: (g, b, 0)),
            pl.BlockSpec((1, BT, IN), lambda g, b: (g, b, 0)),
            pl.BlockSpec((1, BT, 1), lambda g, b: (g, b, 0)),
        ],
        out_shape=[
            jax.ShapeDtypeStruct((G, Bb, ZD), jnp.float32),
            jax.ShapeDtypeStruct((G, Bb, ZD), jnp.float32),
            jax.ShapeDtypeStruct((G, Bb, IN), jnp.float32),
            jax.ShapeDtypeStruct((G, Bb, 1), jnp.int32),
        ],
    )(xf, W1, b1, W2, b2, Wmu, bmu, Wlv, blv, Wd1, bd1, Wd2, bd2, Wd3, bd3)

    xhat = xh_sel.reshape(G, Bb, T, J, D).transpose(1, 2, 0, 3, 4).reshape(Bb, T, G * J, D)
    return mu_sel, lv_sel, xhat, idx[:, :, 0]
